# baseline (speedup 1.0000x reference)
import functools

import jax
import jax.numpy as jnp
from jax import lax
from jax.experimental import pallas as pl
from jax.experimental.pallas import tpu as pltpu
from jax.experimental.pallas import tpu_sc as plsc

B = 16384
D = 64

_info = plsc.get_sparse_core_info()
NC = _info.num_cores
NS = _info.num_subcores
L = _info.num_lanes
NW = NC * NS
BPW = B // NW
G = BPW // L

_mesh = plsc.VectorSubcoreMesh(core_axis_name="c", subcore_axis_name="s")


@functools.partial(
    pl.kernel,
    mesh=_mesh,
    compiler_params=pltpu.CompilerParams(
        needs_layout_passes=False, use_tc_tiling_on_sc=False),
    out_type=(
        jax.ShapeDtypeStruct((B,), jnp.float32),
        jax.ShapeDtypeStruct((B,), jnp.float32),
    ),
    scratch_types=[
        pltpu.VMEM((BPW,), jnp.int32),
        pltpu.VMEM((BPW,), jnp.int32),
        pltpu.VMEM((BPW,), jnp.int32),
        pltpu.VMEM((BPW, D), jnp.float32),
        pltpu.VMEM((BPW, D), jnp.float32),
        pltpu.VMEM((BPW, D), jnp.float32),
        pltpu.VMEM((BPW,), jnp.float32),
        pltpu.VMEM((BPW,), jnp.float32),
        pltpu.SemaphoreType.DMA,
    ],
)
def _bpr_sc(uidx_hbm, pidx_hbm, nidx_hbm, utab_hbm, itab_hbm,
            pos_hbm, neg_hbm,
            uidx_v, pidx_v, nidx_v, urows_v, prows_v, nrows_v,
            pos_v, neg_v, sem):
    wid = lax.axis_index("s") * NC + lax.axis_index("c")
    base = wid * BPW

    pltpu.sync_copy(uidx_hbm.at[pl.ds(base, BPW)], uidx_v)
    pltpu.sync_copy(pidx_hbm.at[pl.ds(base, BPW)], pidx_v)
    pltpu.sync_copy(nidx_hbm.at[pl.ds(base, BPW)], nidx_v)

    cu = pltpu.async_copy(utab_hbm.at[uidx_v], urows_v, sem)
    cp = pltpu.async_copy(itab_hbm.at[pidx_v], prows_v, sem)
    cn = pltpu.async_copy(itab_hbm.at[nidx_v], nrows_v, sem)
    cu.wait()
    cp.wait()
    cn.wait()

    lanes = lax.iota(jnp.int32, L)

    def group_body(g, carry):
        r0 = g * L
        out_p = jnp.zeros((L,), jnp.float32)
        out_n = jnp.zeros((L,), jnp.float32)
        for j in range(L):
            r = r0 + j
            acc_p = jnp.zeros((L,), jnp.float32)
            acc_n = jnp.zeros((L,), jnp.float32)
            for c in range(D // L):
                u = urows_v[r, pl.ds(c * L, L)]
                p = prows_v[r, pl.ds(c * L, L)]
                nn = nrows_v[r, pl.ds(c * L, L)]
                acc_p = acc_p + u * p
                acc_n = acc_n + u * nn
            out_p = jnp.where(lanes == j, jnp.sum(acc_p), out_p)
            out_n = jnp.where(lanes == j, jnp.sum(acc_n), out_n)
        pos_v[pl.ds(r0, L)] = out_p
        neg_v[pl.ds(r0, L)] = out_n
        return carry

    lax.fori_loop(0, BPW // L, group_body, 0)

    pltpu.sync_copy(pos_v, pos_hbm.at[pl.ds(base, BPW)])
    pltpu.sync_copy(neg_v, neg_hbm.at[pl.ds(base, BPW)])


def kernel(user_idx, pos_item_idx, neg_item_idx, user_table, item_table):
    return _bpr_sc(user_idx.astype(jnp.int32),
                   pos_item_idx.astype(jnp.int32),
                   neg_item_idx.astype(jnp.int32),
                   user_table, item_table)


# speedup vs baseline: 1.5542x; 1.5542x over previous
import functools

import jax
import jax.numpy as jnp
from jax import lax
from jax.experimental import pallas as pl
from jax.experimental.pallas import tpu as pltpu
from jax.experimental.pallas import tpu_sc as plsc

B = 16384
D = 64

_info = plsc.get_sparse_core_info()
NC = _info.num_cores
NS = _info.num_subcores
L = _info.num_lanes
NW = NC * NS
BPW = B // NW
CH = 32
NCHUNK = BPW // CH

_mesh = plsc.VectorSubcoreMesh(core_axis_name="c", subcore_axis_name="s")


@functools.partial(
    pl.kernel,
    mesh=_mesh,
    compiler_params=pltpu.CompilerParams(needs_layout_passes=False),
    out_type=(
        jax.ShapeDtypeStruct((B,), jnp.float32),
        jax.ShapeDtypeStruct((B,), jnp.float32),
    ),
    scratch_types=[
        pltpu.VMEM((BPW,), jnp.int32),
        pltpu.VMEM((BPW,), jnp.int32),
        pltpu.VMEM((BPW,), jnp.int32),
        pltpu.VMEM((CH, D), jnp.float32),
        pltpu.VMEM((CH, D), jnp.float32),
        pltpu.VMEM((CH, D), jnp.float32),
        pltpu.VMEM((BPW,), jnp.float32),
        pltpu.VMEM((BPW,), jnp.float32),
        pltpu.SemaphoreType.DMA,
    ],
)
def _bpr_sc(uidx_hbm, pidx_hbm, nidx_hbm, utab_hbm, itab_hbm,
            pos_hbm, neg_hbm,
            uidx_v, pidx_v, nidx_v,
            urows_v, prows_v, nrows_v,
            pos_v, neg_v, sem):
    wid = lax.axis_index("s") * NC + lax.axis_index("c")
    base = wid * BPW

    pltpu.sync_copy(uidx_hbm.at[pl.ds(base, BPW)], uidx_v)
    pltpu.sync_copy(pidx_hbm.at[pl.ds(base, BPW)], pidx_v)
    pltpu.sync_copy(nidx_hbm.at[pl.ds(base, BPW)], nidx_v)

    lanes = lax.iota(jnp.int32, L)

    def chunk_body(ch, carry):
        r0 = ch * CH
        copies = []
        for g in range(CH // L):
            uvec = uidx_v[pl.ds(r0 + g * L, L)]
            pvec = pidx_v[pl.ds(r0 + g * L, L)]
            nvec = nidx_v[pl.ds(r0 + g * L, L)]
            for j in range(L):
                k = g * L + j
                copies.append(pltpu.async_copy(
                    utab_hbm.at[uvec[j]], urows_v.at[k], sem))
                copies.append(pltpu.async_copy(
                    itab_hbm.at[pvec[j]], prows_v.at[k], sem))
                copies.append(pltpu.async_copy(
                    itab_hbm.at[nvec[j]], nrows_v.at[k], sem))
        for c in copies:
            c.wait()
        for g in range(CH // L):
            out_p = jnp.zeros((L,), jnp.float32)
            out_n = jnp.zeros((L,), jnp.float32)
            for j in range(L):
                k = g * L + j
                acc_p = jnp.zeros((L,), jnp.float32)
                acc_n = jnp.zeros((L,), jnp.float32)
                for c in range(D // L):
                    u = urows_v[k, pl.ds(c * L, L)]
                    p = prows_v[k, pl.ds(c * L, L)]
                    nn = nrows_v[k, pl.ds(c * L, L)]
                    acc_p = acc_p + u * p
                    acc_n = acc_n + u * nn
                out_p = jnp.where(lanes == j, jnp.sum(acc_p), out_p)
                out_n = jnp.where(lanes == j, jnp.sum(acc_n), out_n)
            pos_v[pl.ds(r0 + g * L, L)] = out_p
            neg_v[pl.ds(r0 + g * L, L)] = out_n
        return carry

    lax.fori_loop(0, NCHUNK, chunk_body, 0)

    pltpu.sync_copy(pos_v, pos_hbm.at[pl.ds(base, BPW)])
    pltpu.sync_copy(neg_v, neg_hbm.at[pl.ds(base, BPW)])


def kernel(user_idx, pos_item_idx, neg_item_idx, user_table, item_table):
    return _bpr_sc(user_idx.astype(jnp.int32),
                   pos_item_idx.astype(jnp.int32),
                   neg_item_idx.astype(jnp.int32),
                   user_table, item_table)


# speedup vs baseline: 1.5627x; 1.0055x over previous
import functools

import jax
import jax.numpy as jnp
from jax import lax
from jax.experimental import pallas as pl
from jax.experimental.pallas import tpu as pltpu
from jax.experimental.pallas import tpu_sc as plsc

B = 16384
D = 64

_info = plsc.get_sparse_core_info()
NC = _info.num_cores
NS = _info.num_subcores
L = _info.num_lanes
NW = NC * NS
BPW = B // NW
CH = 32
NCHUNK = BPW // CH

_mesh = plsc.VectorSubcoreMesh(core_axis_name="c", subcore_axis_name="s")


@functools.partial(
    pl.kernel,
    mesh=_mesh,
    compiler_params=pltpu.CompilerParams(needs_layout_passes=False),
    out_type=(
        jax.ShapeDtypeStruct((B,), jnp.float32),
        jax.ShapeDtypeStruct((B,), jnp.float32),
    ),
    scratch_types=[
        pltpu.VMEM((BPW,), jnp.int32),
        pltpu.VMEM((BPW,), jnp.int32),
        pltpu.VMEM((BPW,), jnp.int32),
        pltpu.VMEM((CH, D), jnp.float32),
        pltpu.VMEM((CH, D), jnp.float32),
        pltpu.VMEM((CH, D), jnp.float32),
        pltpu.VMEM((BPW,), jnp.float32),
        pltpu.VMEM((BPW,), jnp.float32),
        pltpu.SemaphoreType.DMA,
        pltpu.SemaphoreType.DMA,
        pltpu.SemaphoreType.DMA,
        pltpu.SemaphoreType.DMA,
    ],
)
def _bpr_sc(uidx_hbm, pidx_hbm, nidx_hbm, utab_hbm, itab_hbm,
            pos_hbm, neg_hbm,
            uidx_v, pidx_v, nidx_v,
            urows_v, prows_v, nrows_v,
            pos_v, neg_v, sem, semb, semc, semd):
    wid = lax.axis_index("s") * NC + lax.axis_index("c")
    base = wid * BPW

    pltpu.sync_copy(uidx_hbm.at[pl.ds(base, BPW)], uidx_v)
    pltpu.sync_copy(pidx_hbm.at[pl.ds(base, BPW)], pidx_v)
    pltpu.sync_copy(nidx_hbm.at[pl.ds(base, BPW)], nidx_v)

    lanes = lax.iota(jnp.int32, L)

    def chunk_body(ch, carry):
        r0 = ch * CH
        sems = (sem, semb, semc, semd)
        copies = []
        for g in range(CH // L):
            uvec = uidx_v[pl.ds(r0 + g * L, L)]
            pvec = pidx_v[pl.ds(r0 + g * L, L)]
            nvec = nidx_v[pl.ds(r0 + g * L, L)]
            for j in range(L):
                k = g * L + j
                copies.append(pltpu.async_copy(
                    utab_hbm.at[uvec[j]], urows_v.at[k], sems[k % 4]))
                copies.append(pltpu.async_copy(
                    itab_hbm.at[pvec[j]], prows_v.at[k], sems[(k + 1) % 4]))
                copies.append(pltpu.async_copy(
                    itab_hbm.at[nvec[j]], nrows_v.at[k], sems[(k + 2) % 4]))
        for c in copies:
            c.wait()
        for g in range(CH // L):
            out_p = jnp.zeros((L,), jnp.float32)
            out_n = jnp.zeros((L,), jnp.float32)
            for j in range(L):
                k = g * L + j
                acc_p = jnp.zeros((L,), jnp.float32)
                acc_n = jnp.zeros((L,), jnp.float32)
                for c in range(D // L):
                    u = urows_v[k, pl.ds(c * L, L)]
                    p = prows_v[k, pl.ds(c * L, L)]
                    nn = nrows_v[k, pl.ds(c * L, L)]
                    acc_p = acc_p + u * p
                    acc_n = acc_n + u * nn
                out_p = jnp.where(lanes == j, jnp.sum(acc_p), out_p)
                out_n = jnp.where(lanes == j, jnp.sum(acc_n), out_n)
            pos_v[pl.ds(r0 + g * L, L)] = out_p
            neg_v[pl.ds(r0 + g * L, L)] = out_n
        return carry

    lax.fori_loop(0, NCHUNK, chunk_body, 0)

    pltpu.sync_copy(pos_v, pos_hbm.at[pl.ds(base, BPW)])
    pltpu.sync_copy(neg_v, neg_hbm.at[pl.ds(base, BPW)])


def kernel(user_idx, pos_item_idx, neg_item_idx, user_table, item_table):
    return _bpr_sc(user_idx.astype(jnp.int32),
                   pos_item_idx.astype(jnp.int32),
                   neg_item_idx.astype(jnp.int32),
                   user_table, item_table)
